# baseline (speedup 1.0000x reference)
import dataclasses
import functools

import jax
import jax.numpy as jnp
from jax import lax
from jax.experimental import pallas as pl
from jax.experimental.pallas import tpu as pltpu
from jax.experimental.pallas import tpu_sc as plsc

D = 128
L = 16
NC = 2
NS = 16
NW = NC * NS
CHUNK = 128


def _layernorm_token(rows_v, pos_v, g_v, b_v, t):
    nj = D // L
    acc = jnp.zeros((L,), jnp.float32)
    acc2 = jnp.zeros((L,), jnp.float32)
    vs = []
    for j in range(nj):
        v = rows_v[t, pl.ds(j * L, L)] + pos_v[t, pl.ds(j * L, L)]
        vs.append(v)
        acc = acc + v
        acc2 = acc2 + v * v
    s1 = jnp.sum(acc)
    s2 = jnp.sum(acc2)
    mean = s1 * (1.0 / D)
    var = s2 * (1.0 / D) - mean * mean
    xv = jnp.full((L,), var + 1e-5, jnp.float32)
    bits = lax.bitcast_convert_type(xv, jnp.int32)
    bits = 0x5F3759DF - lax.shift_right_arithmetic(bits, 1)
    y = lax.bitcast_convert_type(bits, jnp.float32)
    for _ in range(3):
        y = y * (1.5 - 0.5 * xv * y * y)
    mv = jnp.full((L,), mean, jnp.float32)
    for j in range(nj):
        sl = pl.ds(j * L, L)
        rows_v[t, sl] = (vs[j] - mv) * y * g_v[sl] + b_v[sl]


def kernel(x, table, pos, gamma, beta):
    B, S = x.shape
    T = B * S
    t_per_w = T // NW
    n_chunks = t_per_w // CHUNK

    mesh = plsc.VectorSubcoreMesh(core_axis_name="c", subcore_axis_name="s")
    cp = pltpu.CompilerParams()
    if "needs_layout_passes" in pltpu.CompilerParams.__dataclass_fields__:
        cp = dataclasses.replace(cp, needs_layout_passes=False)

    @functools.partial(
        pl.kernel,
        mesh=mesh,
        out_type=jax.ShapeDtypeStruct((T, D), jnp.float32),
        scratch_types=[
            pltpu.VMEM((CHUNK,), jnp.int32),
            pltpu.VMEM((CHUNK, D), jnp.float32),
            pltpu.VMEM((CHUNK, D), jnp.float32),
            pltpu.VMEM((D,), jnp.float32),
            pltpu.VMEM((D,), jnp.float32),
            pltpu.SemaphoreType.DMA,
        ],
        compiler_params=cp,
    )
    def sc_embed(x_hbm, tab_hbm, pos_hbm, g_hbm, b_hbm, out_hbm,
                 idx_v, rows_v, pos_v, g_v, b_v, sem):
        wid = lax.axis_index("s") * NC + lax.axis_index("c")
        base0 = wid * t_per_w
        pltpu.sync_copy(g_hbm, g_v)
        pltpu.sync_copy(b_hbm, b_v)
        for ci in range(n_chunks):
            base = base0 + ci * CHUNK
            s0 = lax.rem(base, S)
            pltpu.sync_copy(x_hbm.at[pl.ds(base, CHUNK)], idx_v)
            pltpu.async_copy(tab_hbm.at[idx_v], rows_v, sem).wait()
            pltpu.sync_copy(pos_hbm.at[pl.ds(s0, CHUNK)], pos_v)

            @pl.loop(0, CHUNK)
            def _(t):
                _layernorm_token(rows_v, pos_v, g_v, b_v, t)

            pltpu.sync_copy(rows_v, out_hbm.at[pl.ds(base, CHUNK)])

    out = sc_embed(x.reshape(T), table, pos, gamma, beta)
    return out.reshape(B, S, D)


# speedup vs baseline: 2.6287x; 2.6287x over previous
import dataclasses
import functools

import jax
import jax.numpy as jnp
from jax import lax
from jax.experimental import pallas as pl
from jax.experimental.pallas import tpu as pltpu
from jax.experimental.pallas import tpu_sc as plsc

D = 128
L = 16
NC = 2
NS = 16
NW = NC * NS
CHUNK = 128
NJ = D // L


def _ln_token(rows_v, pos_v, out_v, gs, bs, t):
    acc = jnp.zeros((L,), jnp.float32)
    acc2 = jnp.zeros((L,), jnp.float32)
    vs = []
    for j in range(NJ):
        v = rows_v[t, pl.ds(j * L, L)] + pos_v[t, pl.ds(j * L, L)]
        vs.append(v)
        acc = acc + v
        acc2 = acc2 + v * v
    s1 = jnp.sum(acc)
    s2 = jnp.sum(acc2)
    mean = s1 * (1.0 / D)
    var = s2 * (1.0 / D) - mean * mean
    xv = jnp.full((L,), var + 1e-5, jnp.float32)
    bits = lax.bitcast_convert_type(xv, jnp.int32)
    bits = 0x5F3759DF - lax.shift_right_arithmetic(bits, 1)
    y = lax.bitcast_convert_type(bits, jnp.float32)
    for _ in range(3):
        y = y * (1.5 - 0.5 * xv * y * y)
    mv = jnp.full((L,), mean, jnp.float32)
    for j in range(NJ):
        out_v[t, pl.ds(j * L, L)] = (vs[j] - mv) * y * gs[j] + bs[j]


def kernel(x, table, pos, gamma, beta):
    B, S = x.shape
    T = B * S
    t_per_w = T // NW
    n_chunks = t_per_w // CHUNK

    mesh = plsc.VectorSubcoreMesh(core_axis_name="c", subcore_axis_name="s")
    cp = pltpu.CompilerParams()
    if "needs_layout_passes" in pltpu.CompilerParams.__dataclass_fields__:
        cp = dataclasses.replace(cp, needs_layout_passes=False)

    vmem = pltpu.VMEM
    @functools.partial(
        pl.kernel,
        mesh=mesh,
        out_type=jax.ShapeDtypeStruct((T, D), jnp.float32),
        scratch_types=[
            vmem((2, CHUNK), jnp.int32),
            vmem((2, CHUNK, D), jnp.float32),
            vmem((2, CHUNK, D), jnp.float32),
            vmem((2, CHUNK, D), jnp.float32),
            vmem((D,), jnp.float32),
            vmem((D,), jnp.float32),
            pltpu.SemaphoreType.DMA,
            pltpu.SemaphoreType.DMA,
            pltpu.SemaphoreType.DMA,
            pltpu.SemaphoreType.DMA,
            pltpu.SemaphoreType.DMA,
            pltpu.SemaphoreType.DMA,
            pltpu.SemaphoreType.DMA,
        ],
        compiler_params=cp,
    )
    def sc_embed(x_hbm, tab_hbm, pos_hbm, g_hbm, b_hbm, out_hbm,
                 idx_v, rows_v, pos_v, out_v, g_v, b_v,
                 sg0, sg1, sp0, sp1, so0, so1, sm):
        wid = lax.axis_index("s") * NC + lax.axis_index("c")
        base0 = wid * t_per_w
        pltpu.sync_copy(g_hbm, g_v)
        pltpu.sync_copy(b_hbm, b_v)
        gs = [g_v[pl.ds(j * L, L)] for j in range(NJ)]
        bs = [b_v[pl.ds(j * L, L)] for j in range(NJ)]
        sg = [sg0, sg1]
        sp = [sp0, sp1]
        so = [so0, so1]

        def issue(ci, buf):
            base = base0 + ci * CHUNK
            s0 = lax.rem(base, S)
            pltpu.sync_copy(x_hbm.at[pl.ds(base, CHUNK)], idx_v.at[buf])
            g_cp = pltpu.async_copy(tab_hbm.at[idx_v.at[buf]], rows_v.at[buf], sg[buf])
            p_cp = pltpu.async_copy(pos_hbm.at[pl.ds(s0, CHUNK)], pos_v.at[buf], sp[buf])
            return g_cp, p_cp

        copies = {0: issue(0, 0)}
        out_copies = {}
        for ci in range(n_chunks):
            cur = ci % 2
            if ci + 1 < n_chunks:
                copies[ci + 1] = issue(ci + 1, 1 - cur)
            g_cp, p_cp = copies.pop(ci)
            g_cp.wait()
            p_cp.wait()
            if ci - 2 in out_copies:
                out_copies.pop(ci - 2).wait()

            @pl.loop(0, CHUNK)
            def _(t):
                _ln_token(rows_v.at[cur], pos_v.at[cur], out_v.at[cur], gs, bs, t)

            base = base0 + ci * CHUNK
            out_copies[ci] = pltpu.async_copy(
                out_v.at[cur], out_hbm.at[pl.ds(base, CHUNK)], so[cur])
        for c in out_copies.values():
            c.wait()

    out = sc_embed(x.reshape(T), table, pos, gamma, beta)
    return out.reshape(B, S, D)


# speedup vs baseline: 2.6388x; 1.0039x over previous
import dataclasses
import functools

import jax
import jax.numpy as jnp
from jax import lax
from jax.experimental import pallas as pl
from jax.experimental.pallas import tpu as pltpu
from jax.experimental.pallas import tpu_sc as plsc

D = 128
L = 16
NC = 2
NS = 16
NW = NC * NS
CHUNK = 128
NJ = D // L


def _bcast_last(v):
    idx = lax.full((L,), L - 1, jnp.int32)
    dnums = lax.GatherDimensionNumbers(
        offset_dims=(), collapsed_slice_dims=(0,), start_index_map=(0,))
    return lax.gather(v, idx[:, None], dnums, slice_sizes=(1,),
                      mode=lax.GatherScatterMode.PROMISE_IN_BOUNDS)


def _ln_token(rows_v, pos_v, out_v, gs, bs, t):
    acc = jnp.zeros((L,), jnp.float32)
    acc2 = jnp.zeros((L,), jnp.float32)
    vs = []
    for j in range(NJ):
        v = rows_v[t, pl.ds(j * L, L)] + pos_v[t, pl.ds(j * L, L)]
        vs.append(v)
        acc = acc + v
        acc2 = acc2 + v * v
    mv = _bcast_last(jnp.cumsum(acc)) * (1.0 / D)
    s2v = _bcast_last(jnp.cumsum(acc2)) * (1.0 / D)
    xv = s2v - mv * mv + 1e-5
    bits = lax.bitcast_convert_type(xv, jnp.int32)
    bits = 0x5F3759DF - lax.shift_right_arithmetic(bits, 1)
    y = lax.bitcast_convert_type(bits, jnp.float32)
    for _ in range(3):
        y = y * (1.5 - 0.5 * xv * y * y)
    for j in range(NJ):
        out_v[t, pl.ds(j * L, L)] = (vs[j] - mv) * y * gs[j] + bs[j]


def kernel(x, table, pos, gamma, beta):
    B, S = x.shape
    T = B * S
    t_per_w = T // NW
    n_chunks = t_per_w // CHUNK

    mesh = plsc.VectorSubcoreMesh(core_axis_name="c", subcore_axis_name="s")
    cp = pltpu.CompilerParams()
    if "needs_layout_passes" in pltpu.CompilerParams.__dataclass_fields__:
        cp = dataclasses.replace(cp, needs_layout_passes=False)

    vmem = pltpu.VMEM
    @functools.partial(
        pl.kernel,
        mesh=mesh,
        out_type=jax.ShapeDtypeStruct((T, D), jnp.float32),
        scratch_types=[
            vmem((2, CHUNK), jnp.int32),
            vmem((2, CHUNK, D), jnp.float32),
            vmem((2, CHUNK, D), jnp.float32),
            vmem((2, CHUNK, D), jnp.float32),
            vmem((D,), jnp.float32),
            vmem((D,), jnp.float32),
            pltpu.SemaphoreType.DMA,
            pltpu.SemaphoreType.DMA,
            pltpu.SemaphoreType.DMA,
            pltpu.SemaphoreType.DMA,
            pltpu.SemaphoreType.DMA,
            pltpu.SemaphoreType.DMA,
            pltpu.SemaphoreType.DMA,
        ],
        compiler_params=cp,
    )
    def sc_embed(x_hbm, tab_hbm, pos_hbm, g_hbm, b_hbm, out_hbm,
                 idx_v, rows_v, pos_v, out_v, g_v, b_v,
                 sg0, sg1, sp0, sp1, so0, so1, sm):
        wid = lax.axis_index("s") * NC + lax.axis_index("c")
        base0 = wid * t_per_w
        pltpu.sync_copy(g_hbm, g_v)
        pltpu.sync_copy(b_hbm, b_v)
        gs = [g_v[pl.ds(j * L, L)] for j in range(NJ)]
        bs = [b_v[pl.ds(j * L, L)] for j in range(NJ)]
        sg = [sg0, sg1]
        sp = [sp0, sp1]
        so = [so0, so1]

        def issue(ci, buf):
            base = base0 + ci * CHUNK
            s0 = lax.rem(base, S)
            pltpu.sync_copy(x_hbm.at[pl.ds(base, CHUNK)], idx_v.at[buf])
            g_cp = pltpu.async_copy(tab_hbm.at[idx_v.at[buf]], rows_v.at[buf], sg[buf])
            p_cp = pltpu.async_copy(pos_hbm.at[pl.ds(s0, CHUNK)], pos_v.at[buf], sp[buf])
            return g_cp, p_cp

        copies = {0: issue(0, 0)}
        out_copies = {}
        for ci in range(n_chunks):
            cur = ci % 2
            if ci + 1 < n_chunks:
                copies[ci + 1] = issue(ci + 1, 1 - cur)
            g_cp, p_cp = copies.pop(ci)
            g_cp.wait()
            p_cp.wait()
            if ci - 2 in out_copies:
                out_copies.pop(ci - 2).wait()

            @plsc.parallel_loop(0, CHUNK, 1, unroll=2)
            def _(t):
                _ln_token(rows_v.at[cur], pos_v.at[cur], out_v.at[cur], gs, bs, t)

            base = base0 + ci * CHUNK
            out_copies[ci] = pltpu.async_copy(
                out_v.at[cur], out_hbm.at[pl.ds(base, CHUNK)], so[cur])
        for c in out_copies.values():
            c.wait()

    out = sc_embed(x.reshape(T), table, pos, gamma, beta)
    return out.reshape(B, S, D)
